# initial kernel scaffold (unmeasured)
import jax
import jax.numpy as jnp
from jax import lax
from jax.experimental import pallas as pl
from jax.experimental.pallas import tpu as pltpu


def kernel(Q, K, V):
    b, kv, h, d = K.shape
    scale = d ** -0.5

    def body(q_ref, k_ref, v_ref, o_ref,
             o_scr, st_scr, o_rcv, st_rcv, send_sems, recv_sems):
        i = pl.program_id(0)
        my_x = lax.axis_index("x")
        my_y = lax.axis_index("y")
        my_z = lax.axis_index("z")
        peer = (1 - my_x, my_y, my_z)

        @pl.when(i == 0)
        def _():
            barrier = pltpu.get_barrier_semaphore()
            pl.semaphore_signal(
                barrier, inc=1, device_id=peer,
                device_id_type=pl.DeviceIdType.MESH,
            )
            pl.semaphore_wait(barrier, 1)

        q = q_ref[0, 0]
        k = k_ref[0]
        v = v_ref[0]
        s = jnp.sum(q[None, :, :] * k, axis=-1) * scale
        m = jnp.max(s, axis=0, keepdims=True)
        p = jnp.exp(s - m)
        l = jnp.sum(p, axis=0, keepdims=True)
        o = jnp.sum(p[:, :, None] * v, axis=0)

        o_scr[pl.ds(i, 1)] = o[None]
        st_scr[pl.ds(i, 1), :] = m
        st_scr[pl.ds(b + i, 1), :] = l

        @pl.when(i == b - 1)
        def _():
            copy_o = pltpu.make_async_remote_copy(
                src_ref=o_scr, dst_ref=o_rcv,
                send_sem=send_sems.at[0], recv_sem=recv_sems.at[0],
                device_id=peer, device_id_type=pl.DeviceIdType.MESH,
            )
            copy_s = pltpu.make_async_remote_copy(
                src_ref=st_scr, dst_ref=st_rcv,
                send_sem=send_sems.at[1], recv_sem=recv_sems.at[1],
                device_id=peer, device_id_type=pl.DeviceIdType.MESH,
            )
            copy_o.start()
            copy_s.start()
            copy_o.wait()
            copy_s.wait()

            m_loc = st_scr[pl.ds(0, b), :]
            l_loc = st_scr[pl.ds(b, b), :]
            m_rem = st_rcv[pl.ds(0, b), :]
            l_rem = st_rcv[pl.ds(b, b), :]
            m_new = jnp.maximum(m_loc, m_rem)
            a_loc = jnp.exp(m_loc - m_new)
            a_rem = jnp.exp(m_rem - m_new)
            l_new = l_loc * a_loc + l_rem * a_rem
            o_comb = (
                o_scr[...] * a_loc[:, :, None]
                + o_rcv[...] * a_rem[:, :, None]
            ) / l_new[:, :, None]
            o_ref[...] = o_comb[:, None, :, :]

    return pl.pallas_call(
        body,
        grid=(b,),
        in_specs=[
            pl.BlockSpec((1, 1, h, d), lambda i: (i, 0, 0, 0)),
            pl.BlockSpec((1, kv, h, d), lambda i: (i, 0, 0, 0)),
            pl.BlockSpec((1, kv, h, d), lambda i: (i, 0, 0, 0)),
        ],
        out_specs=pl.BlockSpec((b, 1, h, d), lambda i: (0, 0, 0, 0)),
        out_shape=jax.ShapeDtypeStruct((b, 1, h, d), jnp.float32),
        scratch_shapes=[
            pltpu.VMEM((b, h, d), jnp.float32),
            pltpu.VMEM((2 * b, h), jnp.float32),
            pltpu.VMEM((b, h, d), jnp.float32),
            pltpu.VMEM((2 * b, h), jnp.float32),
            pltpu.SemaphoreType.DMA((2,)),
            pltpu.SemaphoreType.DMA((2,)),
        ],
        compiler_params=pltpu.CompilerParams(
            collective_id=0,
            dimension_semantics=("arbitrary",),
        ),
    )(Q, K, V)


# baseline (device time: 334977 ns/iter reference)
import jax
import jax.numpy as jnp
from jax import lax
from jax.experimental import pallas as pl
from jax.experimental.pallas import tpu as pltpu


def kernel(Q, K, V):
    b, kv, h, d = K.shape
    scale = d ** -0.5

    def body(q_ref, k_ref, v_ref, o_ref,
             o_scr, st_scr, o_rcv, st_rcv, send_sems, recv_sems):
        i = pl.program_id(0)
        my_x = lax.axis_index("x")
        my_y = lax.axis_index("y")
        my_z = lax.axis_index("z")
        peer = (1 - my_x, my_y, my_z)

        @pl.when(i == 0)
        def _():
            barrier = pltpu.get_barrier_semaphore()
            pl.semaphore_signal(
                barrier, inc=1, device_id=peer,
                device_id_type=pl.DeviceIdType.MESH,
            )
            pl.semaphore_wait(barrier, 1)

        q = q_ref[0, 0]
        k = k_ref[0]
        v = v_ref[0]
        s = jnp.sum(q[None, :, :] * k, axis=-1) * scale
        m = jnp.max(s, axis=0, keepdims=True)
        p = jnp.exp(s - m)
        l = jnp.sum(p, axis=0, keepdims=True)
        o = jnp.sum(p[:, :, None] * v, axis=0)

        o_scr[pl.ds(i, 1)] = o[None]
        st_scr[pl.ds(i, 1), :] = m
        st_scr[pl.ds(b + i, 1), :] = l

        @pl.when(i == b - 1)
        def _():
            copy_o = pltpu.make_async_remote_copy(
                src_ref=o_scr, dst_ref=o_rcv,
                send_sem=send_sems.at[0], recv_sem=recv_sems.at[0],
                device_id=peer, device_id_type=pl.DeviceIdType.MESH,
            )
            copy_s = pltpu.make_async_remote_copy(
                src_ref=st_scr, dst_ref=st_rcv,
                send_sem=send_sems.at[1], recv_sem=recv_sems.at[1],
                device_id=peer, device_id_type=pl.DeviceIdType.MESH,
            )
            copy_o.start()
            copy_s.start()
            copy_o.wait()
            copy_s.wait()

            m_loc = st_scr[pl.ds(0, b), :]
            l_loc = st_scr[pl.ds(b, b), :]
            m_rem = st_rcv[pl.ds(0, b), :]
            l_rem = st_rcv[pl.ds(b, b), :]
            m_new = jnp.maximum(m_loc, m_rem)
            a_loc = jnp.exp(m_loc - m_new)
            a_rem = jnp.exp(m_rem - m_new)
            l_new = l_loc * a_loc + l_rem * a_rem
            o_comb = (
                o_scr[...] * a_loc[:, :, None]
                + o_rcv[...] * a_rem[:, :, None]
            ) / l_new[:, :, None]
            o_ref[...] = o_comb[:, None, :, :]

    return pl.pallas_call(
        body,
        grid=(b,),
        in_specs=[
            pl.BlockSpec((1, 1, h, d), lambda i: (i, 0, 0, 0)),
            pl.BlockSpec((1, kv, h, d), lambda i: (i, 0, 0, 0)),
            pl.BlockSpec((1, kv, h, d), lambda i: (i, 0, 0, 0)),
        ],
        out_specs=pl.BlockSpec((b, 1, h, d), lambda i: (0, 0, 0, 0)),
        out_shape=jax.ShapeDtypeStruct((b, 1, h, d), jnp.float32),
        scratch_shapes=[
            pltpu.VMEM((b, h, d), jnp.float32),
            pltpu.VMEM((2 * b, h), jnp.float32),
            pltpu.VMEM((b, h, d), jnp.float32),
            pltpu.VMEM((2 * b, h), jnp.float32),
            pltpu.SemaphoreType.DMA((2,)),
            pltpu.SemaphoreType.DMA((2,)),
        ],
        compiler_params=pltpu.CompilerParams(
            collective_id=0,
            dimension_semantics=("arbitrary",),
            vmem_limit_bytes=64 * 1024 * 1024,
        ),
    )(Q, K, V)


# device time: 178831 ns/iter; 1.8731x vs baseline; 1.8731x over previous
import jax
import jax.numpy as jnp
from jax import lax
from jax.experimental import pallas as pl
from jax.experimental.pallas import tpu as pltpu


def kernel(Q, K, V):
    b, kv, h, d = K.shape
    hd = h * d
    scale = d ** -0.5

    def body(q_ref, k_ref, v_ref, o_ref,
             o_scr, st_scr, o_rcv, st_rcv, send_sems, recv_sems):
        i = pl.program_id(0)
        my_x = lax.axis_index("x")
        my_y = lax.axis_index("y")
        my_z = lax.axis_index("z")
        peer = (1 - my_x, my_y, my_z)

        @pl.when(i == 0)
        def _():
            barrier = pltpu.get_barrier_semaphore()
            pl.semaphore_signal(
                barrier, inc=1, device_id=peer,
                device_id_type=pl.DeviceIdType.MESH,
            )
            pl.semaphore_wait(barrier, 1)

        eyef = (
            lax.broadcasted_iota(jnp.int32, (h, h), 0)
            == lax.broadcasted_iota(jnp.int32, (h, h), 1)
        ).astype(jnp.float32)
        q = q_ref[0]
        qbd = (q[:, None, :] * eyef[:, :, None]).reshape(h, hd)
        k2 = k_ref[0].astype(jnp.bfloat16)
        s = lax.dot_general(
            qbd.astype(jnp.bfloat16), k2,
            (((1,), (1,)), ((), ())),
            preferred_element_type=jnp.float32,
        ) * scale
        m = jnp.max(s, axis=-1, keepdims=True)
        p = jnp.exp(s - m)
        l = jnp.sum(p, axis=-1, keepdims=True)
        v2 = v_ref[0].astype(jnp.bfloat16)
        r = lax.dot_general(
            p.astype(jnp.bfloat16), v2,
            (((1,), (0,)), ((), ())),
            preferred_element_type=jnp.float32,
        )
        o = jnp.sum(r.reshape(h, h, d) * eyef[:, :, None], axis=1)

        o_scr[pl.ds(i, 1)] = o[None]
        st_scr[pl.ds(i, 1), :] = m.reshape(1, h)
        st_scr[pl.ds(b + i, 1), :] = l.reshape(1, h)

        @pl.when(i == b - 1)
        def _():
            copy_o = pltpu.make_async_remote_copy(
                src_ref=o_scr, dst_ref=o_rcv,
                send_sem=send_sems.at[0], recv_sem=recv_sems.at[0],
                device_id=peer, device_id_type=pl.DeviceIdType.MESH,
            )
            copy_s = pltpu.make_async_remote_copy(
                src_ref=st_scr, dst_ref=st_rcv,
                send_sem=send_sems.at[1], recv_sem=recv_sems.at[1],
                device_id=peer, device_id_type=pl.DeviceIdType.MESH,
            )
            copy_o.start()
            copy_s.start()
            copy_o.wait()
            copy_s.wait()

            m_loc = st_scr[pl.ds(0, b), :]
            l_loc = st_scr[pl.ds(b, b), :]
            m_rem = st_rcv[pl.ds(0, b), :]
            l_rem = st_rcv[pl.ds(b, b), :]
            m_new = jnp.maximum(m_loc, m_rem)
            a_loc = jnp.exp(m_loc - m_new)
            a_rem = jnp.exp(m_rem - m_new)
            l_new = l_loc * a_loc + l_rem * a_rem
            o_comb = (
                o_scr[...] * a_loc[:, :, None]
                + o_rcv[...] * a_rem[:, :, None]
            ) / l_new[:, :, None]
            o_ref[...] = o_comb

    out = pl.pallas_call(
        body,
        grid=(b,),
        in_specs=[
            pl.BlockSpec((1, h, d), lambda i: (i, 0, 0)),
            pl.BlockSpec((1, kv, hd), lambda i: (i, 0, 0)),
            pl.BlockSpec((1, kv, hd), lambda i: (i, 0, 0)),
        ],
        out_specs=pl.BlockSpec((b, h, d), lambda i: (0, 0, 0)),
        out_shape=jax.ShapeDtypeStruct((b, h, d), jnp.float32),
        scratch_shapes=[
            pltpu.VMEM((b, h, d), jnp.float32),
            pltpu.VMEM((2 * b, h), jnp.float32),
            pltpu.VMEM((b, h, d), jnp.float32),
            pltpu.VMEM((2 * b, h), jnp.float32),
            pltpu.SemaphoreType.DMA((2,)),
            pltpu.SemaphoreType.DMA((2,)),
        ],
        compiler_params=pltpu.CompilerParams(
            collective_id=0,
            dimension_semantics=("arbitrary",),
            vmem_limit_bytes=64 * 1024 * 1024,
        ),
    )(Q.reshape(b, h, d), K.reshape(b, kv, hd), V.reshape(b, kv, hd))
    return out.reshape(b, 1, h, d)
